# baseline (speedup 1.0000x reference)
import jax
import jax.numpy as jnp
from jax import lax
from jax.experimental import pallas as pl
from jax.experimental.pallas import tpu as pltpu

_FG = 0.7
_BG = 0.3

_B = 8
_G = 100
_T = 10240
_APAD = 20480
_NT = _APAD // _T
_GPAD = 104


def _body(img_ref, ngt_ref, anch_ref, gt_ref, gtr_ref, out_ref, ov_scr,
          gtmax_scr):
    b = pl.program_id(0)
    p = pl.program_id(1)
    t = pl.program_id(2)

    ax1 = anch_ref[0:1, :]
    ay1 = anch_ref[1:2, :]
    ax2 = anch_ref[2:3, :]
    ay2 = anch_ref[3:4, :]
    w = jnp.floor(img_ref[0, 1])
    h = jnp.floor(img_ref[0, 0])
    keep = (ax1 >= 0.0) & (ay1 >= 0.0) & (ax2 < w) & (ay2 < h)

    gts = gt_ref[0]
    gx1 = gts[:, 0:1]
    gy1 = gts[:, 1:2]
    gx2 = gts[:, 2:3]
    gy2 = gts[:, 3:4]
    glab = gts[:, 4:5]
    ngt = ngt_ref[b]
    gid_col = lax.broadcasted_iota(jnp.int32, (_GPAD, 1), 0)
    valid = gid_col < ngt

    @pl.when(p == 0)
    def _pass1():
        iw = jnp.minimum(gx2, ax2) - jnp.maximum(gx1, ax1) + 1.0
        ih = jnp.minimum(gy2, ay2) - jnp.maximum(gy1, ay1) + 1.0
        inter = jnp.maximum(iw, 0.0) * jnp.maximum(ih, 0.0)
        ga = (gx2 - gx1 + 1.0) * (gy2 - gy1 + 1.0)
        aa = (ax2 - ax1 + 1.0) * (ay2 - ay1 + 1.0)
        union = ga + aa - inter
        iou = inter / union
        ov = jnp.where(valid & keep, iou, -1.0)
        ov_scr[:, pl.ds(t * _T, _T)] = ov
        part = jnp.max(ov, axis=1, keepdims=True)

        @pl.when(t == 0)
        def _():
            gtmax_scr[...] = part

        @pl.when(t != 0)
        def _():
            gtmax_scr[...] = jnp.maximum(gtmax_scr[...], part)

    @pl.when(p == 1)
    def _pass2():
        ov = ov_scr[:, pl.ds(t * _T, _T)]
        gids = lax.broadcasted_iota(jnp.int32, (_GPAD, _T), 0)
        max_ov = jnp.max(ov, axis=0, keepdims=True)
        arg_ov = jnp.min(jnp.where(ov == max_ov, gids, _GPAD),
                         axis=0, keepdims=True)
        gtm = gtmax_scr[...]
        cond = (ov == gtm) & (gtm >= 0.0)
        best = jnp.max(jnp.where(cond, gids, -1), axis=0, keepdims=True)

        assign = jnp.full((1, _T), -1, jnp.int32)
        assign = jnp.where((max_ov >= 0.0) & (max_ov < _BG), 0, assign)
        assign = jnp.where(max_ov >= _FG, arg_ov + 1, assign)
        assign = jnp.where(best >= 0, best + 1, assign)
        pos = assign > 0
        neg = assign == 0
        gidx = jnp.clip(assign - 1, 0, _G - 1)

        sel_f = (gids == gidx).astype(jnp.float32)
        rx1 = gtr_ref[0, 0:1, :]
        ry1 = gtr_ref[0, 1:2, :]
        rx2 = gtr_ref[0, 2:3, :]
        ry2 = gtr_ref[0, 3:4, :]
        rlab = gtr_ref[0, 4:5, :]
        rw = rx2 - rx1 + 1.0
        rh = ry2 - ry1 + 1.0
        rcx = rx1 + 0.5 * rw
        rcy = ry1 + 0.5 * rh
        stats = jnp.concatenate([rw, rh, rcx, rcy, rlab], axis=0)
        gathered = jax.lax.dot_general(
            stats, sel_f, (((1,), (0,)), ((), ())),
            precision=jax.lax.Precision.HIGHEST,
            preferred_element_type=jnp.float32)
        sw = gathered[0:1, :]
        sh = gathered[1:2, :]
        scx = gathered[2:3, :]
        scy = gathered[3:4, :]
        slab = gathered[4:5, :]

        ew = ax2 - ax1 + 1.0
        eh = ay2 - ay1 + 1.0
        ecx = ax1 + 0.5 * ew
        ecy = ay1 + 0.5 * eh
        dx = ((scx - ecx) / ew) / 0.1
        dy = ((scy - ecy) / eh) / 0.1
        dwv = jnp.log(sw / ew) / 0.2
        dhv = jnp.log(sh / eh) / 0.2

        cls = jnp.where(pos, slab, jnp.where(neg, 0.0, -1.0))
        cls = jnp.where(keep, cls, -1.0)
        posk = pos & keep
        out_ref[0, 0, 0:1, :] = cls
        out_ref[0, 0, 1:2, :] = jnp.where(posk, dx, 0.0)
        out_ref[0, 0, 2:3, :] = jnp.where(posk, dy, 0.0)
        out_ref[0, 0, 3:4, :] = jnp.where(posk, dwv, 0.0)
        out_ref[0, 0, 4:5, :] = jnp.where(posk, dhv, 0.0)


@jax.jit
def kernel(anchors, gt_boxes, img_info, num_gt_boxes):
    a = anchors.shape[0]
    pad_box = jnp.array([[-100.0, -100.0, -50.0, -50.0]], jnp.float32)
    anch_p = jnp.concatenate(
        [anchors, jnp.broadcast_to(pad_box, (_APAD - a, 4))], axis=0)
    anch_t = anch_p.T
    gt_p = jnp.pad(gt_boxes, ((0, 0), (0, _GPAD - _G), (0, 0)))
    out = pl.pallas_call(
        _body,
        grid=(_B, 2, _NT),
        in_specs=[
            pl.BlockSpec(memory_space=pltpu.SMEM),
            pl.BlockSpec(memory_space=pltpu.SMEM),
            pl.BlockSpec((4, _T), lambda b, p, t: (0, t)),
            pl.BlockSpec((1, _GPAD, 5), lambda b, p, t: (b, 0, 0)),
            pl.BlockSpec((1, 5, _GPAD), lambda b, p, t: (b, 0, 0)),
        ],
        out_specs=pl.BlockSpec((1, 1, 5, _T), lambda b, p, t: (b, p, 0, t)),
        out_shape=jax.ShapeDtypeStruct((_B, 2, 5, _APAD), jnp.float32),
        scratch_shapes=[
            pltpu.VMEM((_GPAD, _APAD), jnp.float32),
            pltpu.VMEM((_GPAD, 1), jnp.float32),
        ],
    )(img_info, num_gt_boxes.astype(jnp.int32), anch_t, gt_p,
      jnp.transpose(gt_p, (0, 2, 1)))
    cls = out[:, 1, 0, :a]
    reg = jnp.transpose(out[:, 1, 1:5, :a], (0, 2, 1))
    return cls, reg


# speedup vs baseline: 1.0679x; 1.0679x over previous
import jax
import jax.numpy as jnp
from jax import lax
from jax.experimental import pallas as pl
from jax.experimental.pallas import tpu as pltpu

_FG = 0.7
_BG = 0.3

_B = 8
_G = 100
_T = 5120
_APAD = 20480
_NT = _APAD // _T
_GPAD = 104


def _body(img_ref, ngt_ref, anch_ref, gt_ref, gtr_ref, out_ref, ov_scr,
          gtmax_scr):
    b = pl.program_id(0)
    p = pl.program_id(1)
    t = pl.program_id(2)

    ax1 = anch_ref[0:1, :]
    ay1 = anch_ref[1:2, :]
    ax2 = anch_ref[2:3, :]
    ay2 = anch_ref[3:4, :]
    w = jnp.floor(img_ref[0, 1])
    h = jnp.floor(img_ref[0, 0])
    keep = (ax1 >= 0.0) & (ay1 >= 0.0) & (ax2 < w) & (ay2 < h)

    gts = gt_ref[0]
    gx1 = gts[:, 0:1]
    gy1 = gts[:, 1:2]
    gx2 = gts[:, 2:3]
    gy2 = gts[:, 3:4]
    glab = gts[:, 4:5]
    ngt = ngt_ref[b]
    gid_col = lax.broadcasted_iota(jnp.int32, (_GPAD, 1), 0)
    valid = gid_col < ngt

    @pl.when(p == 0)
    def _pass1():
        iw = jnp.minimum(gx2, ax2) - jnp.maximum(gx1, ax1) + 1.0
        ih = jnp.minimum(gy2, ay2) - jnp.maximum(gy1, ay1) + 1.0
        inter = jnp.maximum(iw, 0.0) * jnp.maximum(ih, 0.0)
        ga = (gx2 - gx1 + 1.0) * (gy2 - gy1 + 1.0)
        aa = (ax2 - ax1 + 1.0) * (ay2 - ay1 + 1.0)
        union = ga + aa - inter
        iou = inter / union
        ov = jnp.where(valid & keep, iou, -1.0)
        ov_scr[:, pl.ds(t * _T, _T)] = ov
        part = jnp.max(ov, axis=1, keepdims=True)

        @pl.when(t == 0)
        def _():
            gtmax_scr[...] = part

        @pl.when(t != 0)
        def _():
            gtmax_scr[...] = jnp.maximum(gtmax_scr[...], part)

    @pl.when(p == 1)
    def _pass2():
        ov = ov_scr[:, pl.ds(t * _T, _T)]
        gids = lax.broadcasted_iota(jnp.int32, (_GPAD, _T), 0)
        max_ov = jnp.max(ov, axis=0, keepdims=True)
        arg_ov = jnp.min(jnp.where(ov == max_ov, gids, _GPAD),
                         axis=0, keepdims=True)
        gtm = gtmax_scr[...]
        cond = (ov == gtm) & (gtm >= 0.0)
        best = jnp.max(jnp.where(cond, gids, -1), axis=0, keepdims=True)

        assign = jnp.full((1, _T), -1, jnp.int32)
        assign = jnp.where((max_ov >= 0.0) & (max_ov < _BG), 0, assign)
        assign = jnp.where(max_ov >= _FG, arg_ov + 1, assign)
        assign = jnp.where(best >= 0, best + 1, assign)
        pos = assign > 0
        neg = assign == 0
        gidx = jnp.clip(assign - 1, 0, _G - 1)

        sel_f = (gids == gidx).astype(jnp.float32)
        rx1 = gtr_ref[0, 0:1, :]
        ry1 = gtr_ref[0, 1:2, :]
        rx2 = gtr_ref[0, 2:3, :]
        ry2 = gtr_ref[0, 3:4, :]
        rlab = gtr_ref[0, 4:5, :]
        rw = rx2 - rx1 + 1.0
        rh = ry2 - ry1 + 1.0
        rcx = rx1 + 0.5 * rw
        rcy = ry1 + 0.5 * rh
        stats = jnp.concatenate([rw, rh, rcx, rcy, rlab], axis=0)
        gathered = jax.lax.dot_general(
            stats, sel_f, (((1,), (0,)), ((), ())),
            precision=jax.lax.Precision.HIGHEST,
            preferred_element_type=jnp.float32)
        sw = gathered[0:1, :]
        sh = gathered[1:2, :]
        scx = gathered[2:3, :]
        scy = gathered[3:4, :]
        slab = gathered[4:5, :]

        ew = ax2 - ax1 + 1.0
        eh = ay2 - ay1 + 1.0
        ecx = ax1 + 0.5 * ew
        ecy = ay1 + 0.5 * eh
        dx = ((scx - ecx) / ew) / 0.1
        dy = ((scy - ecy) / eh) / 0.1
        dwv = jnp.log(sw / ew) / 0.2
        dhv = jnp.log(sh / eh) / 0.2

        cls = jnp.where(pos, slab, jnp.where(neg, 0.0, -1.0))
        cls = jnp.where(keep, cls, -1.0)
        posk = pos & keep
        out_ref[0, 0, 0:1, :] = cls
        out_ref[0, 0, 1:2, :] = jnp.where(posk, dx, 0.0)
        out_ref[0, 0, 2:3, :] = jnp.where(posk, dy, 0.0)
        out_ref[0, 0, 3:4, :] = jnp.where(posk, dwv, 0.0)
        out_ref[0, 0, 4:5, :] = jnp.where(posk, dhv, 0.0)


@jax.jit
def kernel(anchors, gt_boxes, img_info, num_gt_boxes):
    a = anchors.shape[0]
    pad_box = jnp.array([[-100.0, -100.0, -50.0, -50.0]], jnp.float32)
    anch_p = jnp.concatenate(
        [anchors, jnp.broadcast_to(pad_box, (_APAD - a, 4))], axis=0)
    anch_t = anch_p.T
    gt_p = jnp.pad(gt_boxes, ((0, 0), (0, _GPAD - _G), (0, 0)))
    out = pl.pallas_call(
        _body,
        grid=(_B, 2, _NT),
        in_specs=[
            pl.BlockSpec(memory_space=pltpu.SMEM),
            pl.BlockSpec(memory_space=pltpu.SMEM),
            pl.BlockSpec((4, _T), lambda b, p, t: (0, t)),
            pl.BlockSpec((1, _GPAD, 5), lambda b, p, t: (b, 0, 0)),
            pl.BlockSpec((1, 5, _GPAD), lambda b, p, t: (b, 0, 0)),
        ],
        out_specs=pl.BlockSpec((1, 1, 5, _T), lambda b, p, t: (b, p, 0, t)),
        out_shape=jax.ShapeDtypeStruct((_B, 2, 5, _APAD), jnp.float32),
        scratch_shapes=[
            pltpu.VMEM((_GPAD, _APAD), jnp.float32),
            pltpu.VMEM((_GPAD, 1), jnp.float32),
        ],
    )(img_info, num_gt_boxes.astype(jnp.int32), anch_t, gt_p,
      jnp.transpose(gt_p, (0, 2, 1)))
    cls = out[:, 1, 0, :a]
    reg = jnp.transpose(out[:, 1, 1:5, :a], (0, 2, 1))
    return cls, reg
